# baseline (speedup 1.0000x reference)
import functools

import jax
import jax.numpy as jnp
from jax import lax
from jax.experimental import pallas as pl
from jax.experimental.pallas import tpu as pltpu
from jax.experimental.pallas import tpu_sc as plsc

NC = 2
NS = 16
NW = NC * NS

BATCH = 16384
EMBED = 32
CHUNK = 128
ROWS_PER_W = BATCH // NW
NCH = ROWS_PER_W // CHUNK


def _sc_gather_body(uid, mid, cid, ut, mt, ct, out_u, out_m, out_c,
                    idx_u, idx_m, idx_c, rows_u, rows_m, rows_c, sem):
  wid = lax.axis_index("s") * NC + lax.axis_index("c")
  base = wid * NCH
  pltpu.sync_copy(uid.at[pl.ds(base, NCH)], idx_u)
  pltpu.sync_copy(mid.at[pl.ds(base, NCH)], idx_m)
  pltpu.sync_copy(cid.at[pl.ds(base, NCH)], idx_c)
  copies = []
  for j in range(NCH):
    copies.append(pltpu.async_copy(ut.at[idx_u.at[j]], rows_u.at[j], sem))
    copies.append(pltpu.async_copy(mt.at[idx_m.at[j]], rows_m.at[j], sem))
    copies.append(pltpu.async_copy(ct.at[idx_c.at[j]], rows_c.at[j], sem))
  for cp in copies:
    cp.wait()
  pltpu.sync_copy(rows_u, out_u.at[pl.ds(base, NCH)])
  pltpu.sync_copy(rows_m, out_m.at[pl.ds(base, NCH)])
  pltpu.sync_copy(rows_c, out_c.at[pl.ds(base, NCH)])


@jax.jit
def _sc_gather(uid, mid, cid, ut, mt, ct):
  n = BATCH // CHUNK
  row_t = jax.ShapeDtypeStruct((n, CHUNK, EMBED), jnp.float32)
  mesh = plsc.VectorSubcoreMesh(
      core_axis_name="c", subcore_axis_name="s",
      num_cores=NC, num_subcores=NS)
  fn = pl.kernel(
      _sc_gather_body,
      out_type=(row_t, row_t, row_t),
      mesh=mesh,
      compiler_params=pltpu.CompilerParams(use_tc_tiling_on_sc=False),
      scratch_types=[
          pltpu.VMEM((NCH, CHUNK), jnp.int32),
          pltpu.VMEM((NCH, CHUNK), jnp.int32),
          pltpu.VMEM((NCH, CHUNK), jnp.int32),
          pltpu.VMEM((NCH, CHUNK, EMBED), jnp.float32),
          pltpu.VMEM((NCH, CHUNK, EMBED), jnp.float32),
          pltpu.VMEM((NCH, CHUNK, EMBED), jnp.float32),
          pltpu.SemaphoreType.DMA,
      ],
  )
  return fn(uid.reshape(n, CHUNK), mid.reshape(n, CHUNK),
            cid.reshape(n, CHUNK), ut, mt, ct)


def _mlp_body(ue, me, ce, w1, b1, w2, b2, out):
  h = jnp.dot(ue[...], w1[0:EMBED, :], preferred_element_type=jnp.float32)
  h += jnp.dot(me[...], w1[EMBED:2 * EMBED, :],
               preferred_element_type=jnp.float32)
  h += jnp.dot(ce[...], w1[2 * EMBED:3 * EMBED, :],
               preferred_element_type=jnp.float32)
  h = jnp.maximum(h + b1[...], 0.0)
  out[...] = jnp.sum(h * w2[...], axis=1, keepdims=True) + b2[...]


@functools.partial(jax.jit, static_argnames=("bs",))
def _mlp(ue, me, ce, w1, b1, w2, b2, bs=2048):
  grid = BATCH // bs
  in_block = pl.BlockSpec((bs, EMBED), lambda i: (i, 0))
  full = lambda shape: pl.BlockSpec(shape, lambda i: (0,) * len(shape))
  return pl.pallas_call(
      _mlp_body,
      grid=(grid,),
      in_specs=[in_block, in_block, in_block,
                full((3 * EMBED, 64)), full((1, 64)),
                full((1, 64)), full((1, 1))],
      out_specs=pl.BlockSpec((bs, 1), lambda i: (i, 0)),
      out_shape=jax.ShapeDtypeStruct((BATCH, 1), jnp.float32),
  )(ue, me, ce, w1, b1, w2, b2)


def kernel(user_ids, movie_ids, categories, user_table, movie_table,
           cat_table, W1, b1, W2, b2):
  ue, me, ce = _sc_gather(user_ids.astype(jnp.int32),
                          movie_ids.astype(jnp.int32),
                          categories.astype(jnp.int32),
                          user_table, movie_table, cat_table)
  ue = ue.reshape(BATCH, EMBED)
  me = me.reshape(BATCH, EMBED)
  ce = ce.reshape(BATCH, EMBED)
  return _mlp(ue, me, ce, W1, b1.reshape(1, 64), W2.reshape(1, 64),
              b2.reshape(1, 1))


# speedup vs baseline: 1.5242x; 1.5242x over previous
import functools

import jax
import jax.numpy as jnp
from jax import lax
from jax.experimental import pallas as pl
from jax.experimental.pallas import tpu as pltpu
from jax.experimental.pallas import tpu_sc as plsc

NC = 2
NS = 16
NW = NC * NS

BATCH = 16384
EMBED = 32
SLAB = 8
LANES = 16
ROWS_PER_W = BATCH // NW
NCH = ROWS_PER_W // 128
NCHUNK = ROWS_PER_W // LANES


CHUNK_I = 32
NCHUNKS = ROWS_PER_W // CHUNK_I


def _sc_gather_body(uid, mid, cid, ut, mt, ct, out, idx_v, buf, sem):
  wid = lax.axis_index("s") * NC + lax.axis_index("c")
  base = wid * NCH
  pltpu.sync_copy(uid.at[pl.ds(base, NCH)], idx_v.at[0])
  pltpu.sync_copy(mid.at[pl.ds(base, NCH)], idx_v.at[1])
  pltpu.sync_copy(cid.at[pl.ds(base, NCH)], idx_v.at[2])
  tables = (ut, mt, ct)

  def chunk_body(cc, _):
    j = cc // (128 // LANES)
    col0 = (cc - j * (128 // LANES)) * LANES
    w = [idx_v[t, j, pl.ds(col0, LANES)] for t in range(3)]
    for ii in range(LANES):
      col = col0 + ii
      for t in range(3):
        pltpu.async_copy(tables[t].at[w[t][ii]],
                         buf.at[j, col, pl.ds(t * EMBED, EMBED)], sem)
    for ii in range(LANES):
      for t in range(3):
        pltpu.make_async_copy(
            tables[t].at[0],
            buf.at[0, 0, pl.ds(t * EMBED, EMBED)], sem).wait()
    return 0

  lax.fori_loop(0, NCH * (128 // LANES), chunk_body, 0)
  pltpu.sync_copy(buf, out.at[pl.ds(base, NCH)])


@jax.jit
def _sc_gather(uid, mid, cid, ut, mt, ct):
  n = BATCH // 128
  mesh = plsc.VectorSubcoreMesh(
      core_axis_name="c", subcore_axis_name="s",
      num_cores=NC, num_subcores=NS)
  fn = pl.kernel(
      _sc_gather_body,
      out_type=jax.ShapeDtypeStruct((n, 128, 3 * EMBED), jnp.float32),
      mesh=mesh,
      scratch_types=[
          pltpu.VMEM((3, NCH, 128), jnp.int32),
          pltpu.VMEM((NCH, 128, 3 * EMBED), jnp.float32),
          pltpu.SemaphoreType.DMA,
      ],
  )
  return fn(uid.reshape(n, 128), mid.reshape(n, 128), cid.reshape(n, 128),
            ut, mt, ct)


def _mlp_body(x, w1, b1, w2, b2, out):
  h = jnp.dot(x[...], w1[...], preferred_element_type=jnp.float32)
  h = jnp.maximum(h + b1[...], 0.0)
  out[...] = jnp.sum(h * w2[...], axis=1, keepdims=True) + b2[...]


@functools.partial(jax.jit, static_argnames=("bs",))
def _mlp(x, w1, b1, w2, b2, bs=2048):
  grid = BATCH // bs
  full = lambda shape: pl.BlockSpec(shape, lambda i: (0,) * len(shape))
  return pl.pallas_call(
      _mlp_body,
      grid=(grid,),
      in_specs=[pl.BlockSpec((bs, 3 * EMBED), lambda i: (i, 0)),
                full((3 * EMBED, 64)), full((1, 64)),
                full((1, 64)), full((1, 1))],
      out_specs=pl.BlockSpec((bs, 1), lambda i: (i, 0)),
      out_shape=jax.ShapeDtypeStruct((BATCH, 1), jnp.float32),
  )(x, w1, b1, w2, b2)


def kernel(user_ids, movie_ids, categories, user_table, movie_table,
           cat_table, W1, b1, W2, b2):
  x = _sc_gather(user_ids.astype(jnp.int32), movie_ids.astype(jnp.int32),
                 categories.astype(jnp.int32),
                 user_table, movie_table, cat_table)
  x = x.reshape(BATCH, 3 * EMBED)
  return _mlp(x, W1, b1.reshape(1, 64), W2.reshape(1, 64), b2.reshape(1, 1))
